# baseline (speedup 1.0000x reference)
import functools

import jax
import jax.numpy as jnp
from jax.experimental import pallas as pl
from jax.experimental.pallas import tpu as pltpu

_CHUNK = 128
_CUT = 0.05


def _lr_kernel(mcol_ref, mrow_ref, ei_ref, ej_ref, xj_ref, xi_ref, out_ref,
               wbuf_ref, accy_ref, num_ref, *, bi, bj, batch, nbj):
    i = pl.program_id(0)
    j = pl.program_id(1)
    p = jax.lax.rem(i * (nbj + 1) + j, 2)

    @pl.when((i == 0) & (j == 0))
    def _boot():
        wbuf_ref[1] = jnp.zeros_like(wbuf_ref[1])

    @pl.when(j == 0)
    def _zero_row():
        num_ref[...] = jnp.zeros_like(num_ref)
        accy_ref[...] = jnp.zeros_like(accy_ref)

    w_prev = wbuf_ref[1 - p]
    for b in range(batch):
        accy_ref[b] += jnp.dot(w_prev, xj_ref[b],
                               preferred_element_type=jnp.float32)

    ei = ei_ref[...]
    ej = ej_ref[...]
    ein = ei / jnp.maximum(
        jnp.sqrt(jnp.sum(ei * ei, axis=1, keepdims=True)), 1e-8)
    ejn = ej / jnp.maximum(
        jnp.sqrt(jnp.sum(ej * ej, axis=1, keepdims=True)), 1e-8)
    s = jnp.abs(jax.lax.dot_general(
        ein, ejn, (((1,), (1,)), ((), ())),
        preferred_element_type=jnp.float32))
    live = (j < nbj).astype(jnp.float32)
    mi = mcol_ref[0].astype(jnp.float32) * live
    mj = mrow_ref[0].astype(jnp.float32)
    s = s * (mi * mj)
    ii = i * bi + jax.lax.broadcasted_iota(jnp.int32, (bi, bj), 0)
    jjp = j * bj + jax.lax.broadcasted_iota(jnp.int32, (bi, bj), 1)
    keep = (jnp.abs(ii - jjp) > _CHUNK) & (s > _CUT)
    wbuf_ref[p] = jnp.where(keep, s, 0.0)
    num_ref[...] += jnp.sum(keep.astype(jnp.float32), axis=1, keepdims=True)

    @pl.when(j == nbj)
    def _fin():
        num = num_ref[...]
        xi = xi_ref[...]
        y = accy_ref[...] / jnp.maximum(num, 1.0)[None]
        out_ref[...] = jnp.where((num > 0.0)[None], (xi + y) * 0.5, xi)


@jax.jit
def kernel(x, mask, emb_i_weight, emb_j_weight):
    B, L, D = x.shape
    E = emb_i_weight.shape[1]
    bi = 512 if L % 512 == 0 else 128
    bj = 2048 if L % 2048 == 0 else bi
    nbi = L // bi
    nbj = L // bj
    mask_col = mask.reshape(nbi, bi, 1)
    mask_row = mask.reshape(nbj, 1, bj)
    return pl.pallas_call(
        functools.partial(_lr_kernel, bi=bi, bj=bj, batch=B, nbj=nbj),
        grid=(nbi, nbj + 1),
        in_specs=[
            pl.BlockSpec((1, bi, 1), lambda i, j: (i, 0, 0)),
            pl.BlockSpec((1, 1, bj), lambda i, j: (jnp.minimum(j, nbj - 1), 0, 0)),
            pl.BlockSpec((bi, E), lambda i, j: (i, 0)),
            pl.BlockSpec((bj, E), lambda i, j: (jnp.minimum(j, nbj - 1), 0)),
            pl.BlockSpec((B, bj, D),
                         lambda i, j: (0, jnp.maximum(j, 1) - 1, 0)),
            pl.BlockSpec((B, bi, D), lambda i, j: (0, i, 0)),
        ],
        out_specs=pl.BlockSpec((B, bi, D), lambda i, j: (0, i, 0)),
        out_shape=jax.ShapeDtypeStruct((B, L, D), x.dtype),
        scratch_shapes=[
            pltpu.VMEM((2, bi, bj), jnp.float32),
            pltpu.VMEM((B, bi, D), jnp.float32),
            pltpu.VMEM((bi, 1), jnp.float32),
        ],
        compiler_params=pltpu.CompilerParams(
            dimension_semantics=("arbitrary", "arbitrary")),
    )(mask_col, mask_row, emb_i_weight, emb_j_weight, x, x)


# speedup vs baseline: 1.1044x; 1.1044x over previous
import functools

import jax
import jax.numpy as jnp
from jax.experimental import pallas as pl
from jax.experimental.pallas import tpu as pltpu

_CHUNK = 128
_CUT = 0.05


def _lr_kernel(mcol_ref, mrow_ref, ei_ref, ej_ref, xj_ref, xi_ref, out_ref,
               wbuf_ref, accy_ref, num_ref, *, bi, bj, batch, nbj):
    i = pl.program_id(0)
    j = pl.program_id(1)
    p = jax.lax.rem(i * (nbj + 1) + j, 2)

    @pl.when((i == 0) & (j == 0))
    def _boot():
        wbuf_ref[1] = jnp.zeros_like(wbuf_ref[1])

    @pl.when(j == 0)
    def _zero_row():
        num_ref[...] = jnp.zeros_like(num_ref)
        accy_ref[...] = jnp.zeros_like(accy_ref)

    w_prev = wbuf_ref[1 - p]
    for b in range(batch):
        accy_ref[b] += jnp.dot(w_prev, xj_ref[b],
                               preferred_element_type=jnp.float32)

    ei = ei_ref[...]
    ej = ej_ref[...]
    ein = ei / jnp.maximum(
        jnp.sqrt(jnp.sum(ei * ei, axis=1, keepdims=True)), 1e-8)
    ejn = ej / jnp.maximum(
        jnp.sqrt(jnp.sum(ej * ej, axis=1, keepdims=True)), 1e-8)
    s = jnp.abs(jax.lax.dot_general(
        ein, ejn, (((1,), (1,)), ((), ())),
        preferred_element_type=jnp.float32))
    live = (j < nbj).astype(jnp.float32)
    mi = mcol_ref[0].astype(jnp.float32) * live
    mj = mrow_ref[0].astype(jnp.float32)
    s = s * (mi * mj)
    ii = i * bi + jax.lax.broadcasted_iota(jnp.int32, (bi, bj), 0)
    jjp = j * bj + jax.lax.broadcasted_iota(jnp.int32, (bi, bj), 1)
    keep = (jnp.abs(ii - jjp) > _CHUNK) & (s > _CUT)
    wbuf_ref[p] = jnp.where(keep, s, 0.0).astype(jnp.bfloat16)
    num_ref[...] += jnp.sum(keep.astype(jnp.float32), axis=1, keepdims=True)

    @pl.when(j == nbj)
    def _fin():
        num = num_ref[...]
        xi = xi_ref[...]
        y = accy_ref[...] / jnp.maximum(num, 1.0)[None]
        out_ref[...] = jnp.where((num > 0.0)[None], (xi + y) * 0.5, xi)


@jax.jit
def kernel(x, mask, emb_i_weight, emb_j_weight):
    B, L, D = x.shape
    E = emb_i_weight.shape[1]
    bi = 512 if L % 512 == 0 else 128
    bj = 1024 if L % 1024 == 0 else bi
    nbi = L // bi
    nbj = L // bj
    mask_col = mask.reshape(nbi, bi, 1)
    mask_row = mask.reshape(nbj, 1, bj)
    return pl.pallas_call(
        functools.partial(_lr_kernel, bi=bi, bj=bj, batch=B, nbj=nbj),
        grid=(nbi, nbj + 1),
        in_specs=[
            pl.BlockSpec((1, bi, 1), lambda i, j: (i, 0, 0)),
            pl.BlockSpec((1, 1, bj), lambda i, j: (jnp.minimum(j, nbj - 1), 0, 0)),
            pl.BlockSpec((bi, E), lambda i, j: (i, 0)),
            pl.BlockSpec((bj, E), lambda i, j: (jnp.minimum(j, nbj - 1), 0)),
            pl.BlockSpec((B, bj, D),
                         lambda i, j: (0, jnp.maximum(j, 1) - 1, 0)),
            pl.BlockSpec((B, bi, D), lambda i, j: (0, i, 0)),
        ],
        out_specs=pl.BlockSpec((B, bi, D), lambda i, j: (0, i, 0)),
        out_shape=jax.ShapeDtypeStruct((B, L, D), x.dtype),
        scratch_shapes=[
            pltpu.VMEM((2, bi, bj), jnp.bfloat16),
            pltpu.VMEM((B, bi, D), jnp.float32),
            pltpu.VMEM((bi, 1), jnp.float32),
        ],
        compiler_params=pltpu.CompilerParams(
            dimension_semantics=("arbitrary", "arbitrary")),
    )(mask_col, mask_row, emb_i_weight, emb_j_weight,
      x.astype(jnp.bfloat16), x)


# speedup vs baseline: 1.1239x; 1.0177x over previous
import functools

import jax
import jax.numpy as jnp
from jax.experimental import pallas as pl
from jax.experimental.pallas import tpu as pltpu

_CHUNK = 128
_CUT = 0.05


def _lr_kernel(mcol_ref, mrow_ref, ei_ref, ej_ref, xj_ref, xi_ref, out_ref,
               wbuf_ref, accy_ref, num_ref, *, bi, bj, batch, nbj):
    i = pl.program_id(0)
    j = pl.program_id(1)
    p = jax.lax.rem(i * (nbj + 1) + j, 2)

    @pl.when((i == 0) & (j == 0))
    def _boot():
        wbuf_ref[1] = jnp.zeros_like(wbuf_ref[1])

    @pl.when(j == 0)
    def _zero_row():
        num_ref[...] = jnp.zeros_like(num_ref)
        accy_ref[...] = jnp.zeros_like(accy_ref)

    w_prev = wbuf_ref[1 - p]
    for b in range(batch):
        accy_ref[b] += jnp.dot(w_prev, xj_ref[b],
                               preferred_element_type=jnp.float32)

    ei = ei_ref[...]
    ej = ej_ref[...]
    ein = ei / jnp.maximum(
        jnp.sqrt(jnp.sum(ei * ei, axis=1, keepdims=True)), 1e-8)
    ejn = ej / jnp.maximum(
        jnp.sqrt(jnp.sum(ej * ej, axis=1, keepdims=True)), 1e-8)
    s = jnp.abs(jax.lax.dot_general(
        ein, ejn, (((1,), (1,)), ((), ())),
        preferred_element_type=jnp.float32))
    live = (j < nbj).astype(jnp.float32)
    mi = mcol_ref[0].astype(jnp.float32) * live
    mj = mrow_ref[0].astype(jnp.float32)
    s = s * (mi * mj)
    ii = i * bi + jax.lax.broadcasted_iota(jnp.int32, (bi, bj), 0)
    jjp = j * bj + jax.lax.broadcasted_iota(jnp.int32, (bi, bj), 1)
    keep = (jnp.abs(ii - jjp) > _CHUNK) & (s > _CUT)
    wbuf_ref[p] = jnp.where(keep, s, 0.0)
    num_ref[...] += jnp.sum(keep.astype(jnp.float32), axis=1, keepdims=True)

    @pl.when(j == nbj)
    def _fin():
        num = num_ref[...]
        xi = xi_ref[...]
        y = accy_ref[...] / jnp.maximum(num, 1.0)[None]
        out_ref[...] = jnp.where((num > 0.0)[None], (xi + y) * 0.5, xi)


@jax.jit
def kernel(x, mask, emb_i_weight, emb_j_weight):
    B, L, D = x.shape
    E = emb_i_weight.shape[1]
    bi = 512 if L % 512 == 0 else 128
    bj = 1024 if L % 1024 == 0 else bi
    nbi = L // bi
    nbj = L // bj
    mask_col = mask.reshape(nbi, bi, 1)
    mask_row = mask.reshape(nbj, 1, bj)
    return pl.pallas_call(
        functools.partial(_lr_kernel, bi=bi, bj=bj, batch=B, nbj=nbj),
        grid=(nbi, nbj + 1),
        in_specs=[
            pl.BlockSpec((1, bi, 1), lambda i, j: (i, 0, 0)),
            pl.BlockSpec((1, 1, bj), lambda i, j: (jnp.minimum(j, nbj - 1), 0, 0)),
            pl.BlockSpec((bi, E), lambda i, j: (i, 0)),
            pl.BlockSpec((bj, E), lambda i, j: (jnp.minimum(j, nbj - 1), 0)),
            pl.BlockSpec((B, bj, D),
                         lambda i, j: (0, jnp.maximum(j, 1) - 1, 0)),
            pl.BlockSpec((B, bi, D), lambda i, j: (0, i, 0)),
        ],
        out_specs=pl.BlockSpec((B, bi, D), lambda i, j: (0, i, 0)),
        out_shape=jax.ShapeDtypeStruct((B, L, D), x.dtype),
        scratch_shapes=[
            pltpu.VMEM((2, bi, bj), jnp.float32),
            pltpu.VMEM((B, bi, D), jnp.float32),
            pltpu.VMEM((bi, 1), jnp.float32),
        ],
        compiler_params=pltpu.CompilerParams(
            dimension_semantics=("arbitrary", "arbitrary")),
    )(mask_col, mask_row, emb_i_weight, emb_j_weight, x, x)


# speedup vs baseline: 1.1251x; 1.0011x over previous
import functools

import jax
import jax.numpy as jnp
from jax.experimental import pallas as pl
from jax.experimental.pallas import tpu as pltpu

_CHUNK = 128
_CUT = 0.05


def _lr_kernel(mcol_ref, mrow_ref, ei_ref, ej_ref, xj_ref, xi_ref, out_ref,
               wbuf_ref, accy_ref, num_ref, *, bi, bj, batch, nbj):
    i = pl.program_id(0)
    j = pl.program_id(1)
    p = jax.lax.rem(i * (nbj + 1) + j, 2)

    @pl.when((i == 0) & (j == 0))
    def _boot():
        wbuf_ref[1] = jnp.zeros_like(wbuf_ref[1])

    @pl.when(j == 0)
    def _zero_row():
        num_ref[...] = jnp.zeros_like(num_ref)
        accy_ref[...] = jnp.zeros_like(accy_ref)

    w_prev = wbuf_ref[1 - p]
    for b in range(batch):
        accy_ref[b] += jnp.dot(w_prev, xj_ref[b],
                               preferred_element_type=jnp.float32)

    ei = ei_ref[...]
    ej = ej_ref[...]
    ein = ei / jnp.maximum(
        jnp.sqrt(jnp.sum(ei * ei, axis=1, keepdims=True)), 1e-8)
    ejn = ej / jnp.maximum(
        jnp.sqrt(jnp.sum(ej * ej, axis=1, keepdims=True)), 1e-8)
    s = jnp.abs(jax.lax.dot_general(
        ein, ejn, (((1,), (1,)), ((), ())),
        preferred_element_type=jnp.float32))
    live = (j < nbj).astype(jnp.float32)
    mi = mcol_ref[0].astype(jnp.float32) * live
    mj = mrow_ref[0].astype(jnp.float32)
    s = s * (mi * mj)
    ii = i * bi + jax.lax.broadcasted_iota(jnp.int32, (bi, bj), 0)
    jjp = j * bj + jax.lax.broadcasted_iota(jnp.int32, (bi, bj), 1)
    keep = (jnp.abs(ii - jjp) > _CHUNK) & (s > _CUT)
    wbuf_ref[p] = jnp.where(keep, s, 0.0)
    num_ref[...] += jnp.sum(keep.astype(jnp.float32), axis=1, keepdims=True)

    @pl.when(j == nbj)
    def _fin():
        num = num_ref[...]
        xi = xi_ref[...]
        y = accy_ref[...] / jnp.maximum(num, 1.0)[None]
        out_ref[...] = jnp.where((num > 0.0)[None], (xi + y) * 0.5, xi)


@jax.jit
def kernel(x, mask, emb_i_weight, emb_j_weight):
    B, L, D = x.shape
    E = emb_i_weight.shape[1]
    bi = 512 if L % 512 == 0 else 128
    bj = 1024 if L % 1024 == 0 else bi
    nbi = L // bi
    nbj = L // bj
    mask_col = mask.reshape(nbi, bi, 1)
    mask_row = mask.reshape(nbj, 1, bj)
    return pl.pallas_call(
        functools.partial(_lr_kernel, bi=bi, bj=bj, batch=B, nbj=nbj),
        grid=(nbi, nbj + 1),
        in_specs=[
            pl.BlockSpec((1, bi, 1), lambda i, j: (i, 0, 0)),
            pl.BlockSpec((1, 1, bj), lambda i, j: (jnp.minimum(j, nbj - 1), 0, 0)),
            pl.BlockSpec((bi, E), lambda i, j: (i, 0)),
            pl.BlockSpec((bj, E), lambda i, j: (jnp.minimum(j, nbj - 1), 0)),
            pl.BlockSpec((B, bj, D),
                         lambda i, j: (0, jnp.maximum(j, 1) - 1, 0)),
            pl.BlockSpec((B, bi, D), lambda i, j: (0, i, 0)),
        ],
        out_specs=pl.BlockSpec((B, bi, D), lambda i, j: (0, i, 0)),
        out_shape=jax.ShapeDtypeStruct((B, L, D), x.dtype),
        scratch_shapes=[
            pltpu.VMEM((2, bi, bj), jnp.float32),
            pltpu.VMEM((B, bi, D), jnp.float32),
            pltpu.VMEM((bi, 1), jnp.float32),
        ],
        compiler_params=pltpu.CompilerParams(
            dimension_semantics=("parallel", "arbitrary")),
    )(mask_col, mask_row, emb_i_weight, emb_j_weight, x, x)


# speedup vs baseline: 1.2248x; 1.0886x over previous
import functools

import jax
import jax.numpy as jnp
from jax.experimental import pallas as pl
from jax.experimental.pallas import tpu as pltpu

_CHUNK = 128
_CUT = 0.05


def _lr_kernel(mcol_ref, mrow_ref, ei_ref, ej_ref, xj_ref, xi_ref, out_ref,
               wbuf_ref, accy_ref, num_ref, *, bi, bj, batch, nbj):
    i = pl.program_id(0)
    j = pl.program_id(1)
    p = jax.lax.rem(i * (nbj + 1) + j, 2)

    @pl.when((i == 0) & (j == 0))
    def _boot():
        wbuf_ref[1] = jnp.zeros_like(wbuf_ref[1])

    @pl.when(j == 0)
    def _zero_row():
        num_ref[...] = jnp.zeros_like(num_ref)
        accy_ref[...] = jnp.zeros_like(accy_ref)

    w_prev = wbuf_ref[1 - p]
    for b in range(batch):
        accy_ref[b] += jnp.dot(w_prev, xj_ref[b],
                               preferred_element_type=jnp.float32)

    ei = ei_ref[...]
    ej = ej_ref[...]
    ein = ei / jnp.maximum(
        jnp.sqrt(jnp.sum(ei * ei, axis=1, keepdims=True)), 1e-8)
    ejn = ej / jnp.maximum(
        jnp.sqrt(jnp.sum(ej * ej, axis=1, keepdims=True)), 1e-8)
    s = jnp.abs(jax.lax.dot_general(
        ein, ejn, (((1,), (1,)), ((), ())),
        preferred_element_type=jnp.float32))
    live = (j < nbj).astype(jnp.float32)
    mi = mcol_ref[0].astype(jnp.float32) * live
    mj = mrow_ref[0].astype(jnp.float32)
    s = s * (mi * mj)
    ii = i * bi + jax.lax.broadcasted_iota(jnp.int32, (bi, bj), 0)
    jjp = j * bj + jax.lax.broadcasted_iota(jnp.int32, (bi, bj), 1)
    keep = (jnp.abs(ii - jjp) > _CHUNK) & (s > _CUT)
    wbuf_ref[p] = jnp.where(keep, s, 0.0).astype(jnp.bfloat16)
    num_ref[...] += jnp.sum(keep.astype(jnp.float32), axis=1, keepdims=True)

    @pl.when(j == nbj)
    def _fin():
        num = num_ref[...]
        xi = xi_ref[...]
        y = accy_ref[...] / jnp.maximum(num, 1.0)[None]
        out_ref[...] = jnp.where((num > 0.0)[None], (xi + y) * 0.5, xi)


@jax.jit
def kernel(x, mask, emb_i_weight, emb_j_weight):
    B, L, D = x.shape
    E = emb_i_weight.shape[1]
    bi = 1024 if L % 1024 == 0 else 128
    bj = 1024 if L % 1024 == 0 else bi
    nbi = L // bi
    nbj = L // bj
    mask_col = mask.reshape(nbi, bi, 1)
    mask_row = mask.reshape(nbj, 1, bj)
    return pl.pallas_call(
        functools.partial(_lr_kernel, bi=bi, bj=bj, batch=B, nbj=nbj),
        grid=(nbi, nbj + 1),
        in_specs=[
            pl.BlockSpec((1, bi, 1), lambda i, j: (i, 0, 0)),
            pl.BlockSpec((1, 1, bj), lambda i, j: (jnp.minimum(j, nbj - 1), 0, 0)),
            pl.BlockSpec((bi, E), lambda i, j: (i, 0)),
            pl.BlockSpec((bj, E), lambda i, j: (jnp.minimum(j, nbj - 1), 0)),
            pl.BlockSpec((B, bj, D),
                         lambda i, j: (0, jnp.maximum(j, 1) - 1, 0)),
            pl.BlockSpec((B, bi, D), lambda i, j: (0, i, 0)),
        ],
        out_specs=pl.BlockSpec((B, bi, D), lambda i, j: (0, i, 0)),
        out_shape=jax.ShapeDtypeStruct((B, L, D), x.dtype),
        scratch_shapes=[
            pltpu.VMEM((2, bi, bj), jnp.bfloat16),
            pltpu.VMEM((B, bi, D), jnp.float32),
            pltpu.VMEM((bi, 1), jnp.float32),
        ],
        compiler_params=pltpu.CompilerParams(
            dimension_semantics=("parallel", "arbitrary")),
    )(mask_col, mask_row, emb_i_weight, emb_j_weight, x, x)
